# baseline (speedup 1.0000x reference)
import jax
import jax.numpy as jnp
from jax import lax
from jax.experimental import pallas as pl
from jax.experimental.pallas import tpu as pltpu
from jax.experimental.pallas import tpu_sc as plsc

BATCH = 4
NUM_TOKENS = 8192
EMBED_DIM = 1024

NC = 2
NS = 16
NW = NC * NS
L = 16

TOK_PER_TILE = NUM_TOKENS // NW
T_CH = 16
CH = T_CH * EMBED_DIM
N_CH = TOK_PER_TILE // T_CH
NBUF = 5
LOOKAHEAD = 2
N_IT = N_CH * BATCH
UNROLL = 8


def _body(tok_hbm, tbl_hbm, out_hbm, *scratch):
    tbl_v = scratch[0:2]
    tok_v = scratch[2:2 + NBUF]
    s_tbl = scratch[2 + NBUF:4 + NBUF]
    s_in = scratch[4 + NBUF:4 + 2 * NBUF]
    s_out = scratch[4 + 2 * NBUF:4 + 3 * NBUF]

    wid = lax.axis_index("s") * NC + lax.axis_index("c")
    t_base = wid * TOK_PER_TILE

    def row0(ci):
        return t_base + ci * T_CH

    def start_tbl(ci):
        return pltpu.async_copy(
            tbl_hbm.at[pl.ds(row0(ci), T_CH), :], tbl_v[ci % 2], s_tbl[ci % 2])

    def start_in(g):
        ci, b = divmod(g, BATCH)
        return pltpu.async_copy(
            tok_hbm.at[b, pl.ds(row0(ci), T_CH), :], tok_v[g % NBUF], s_in[g % NBUF])

    def start_out(g):
        ci, b = divmod(g, BATCH)
        return pltpu.async_copy(
            tok_v[g % NBUF], out_hbm.at[b, pl.ds(row0(ci), T_CH), :], s_out[g % NBUF])

    tbl_d = {ci: start_tbl(ci) for ci in range(min(2, N_CH))}
    in_d = {g: start_in(g) for g in range(min(NBUF, N_IT))}
    out_d = {}

    for g in range(N_IT):
        ci, b = divmod(g, BATCH)
        tb = ci % 2
        buf = g % NBUF

        if b == 0:
            tbl_d.pop(ci).wait()

        h = g + LOOKAHEAD
        if NBUF <= h < N_IT:
            out_d.pop(h - NBUF).wait()
            in_d[h] = start_in(h)

        in_d.pop(g).wait()

        tok_b = tok_v[buf]
        tbl_b = tbl_v[tb]

        @plsc.parallel_loop(0, CH, L, unroll=UNROLL)
        def add_body(i):
            r = lax.shift_right_logical(i, 10)
            c = pl.multiple_of(lax.bitwise_and(i, EMBED_DIM - 1), L)
            tok_b[r, pl.ds(c, L)] = tok_b[r, pl.ds(c, L)] + tbl_b[r, pl.ds(c, L)]

        out_d[g] = start_out(g)

        if b == BATCH - 1 and ci + 2 < N_CH:
            tbl_d[ci + 2] = start_tbl(ci + 2)

    for od in out_d.values():
        od.wait()


@jax.jit
def _pos_add(encoded_tokens, position_table):
    mesh = plsc.VectorSubcoreMesh(core_axis_name="c", subcore_axis_name="s")
    return pl.kernel(
        _body,
        out_type=jax.ShapeDtypeStruct((BATCH, NUM_TOKENS, EMBED_DIM), jnp.float32),
        mesh=mesh,
        compiler_params=pltpu.CompilerParams(
            use_tc_tiling_on_sc=True, skip_device_barrier=True),
        scratch_types=(
            [pltpu.VMEM((T_CH, EMBED_DIM), jnp.float32)] * 2
            + [pltpu.VMEM((T_CH, EMBED_DIM), jnp.float32)] * NBUF
            + [pltpu.SemaphoreType.DMA] * (2 + 2 * NBUF)
        ),
    )(encoded_tokens, position_table)


def kernel(encoded_tokens, position_table):
    return _pos_add(encoded_tokens, position_table)


# speedup vs baseline: 1.0100x; 1.0100x over previous
import jax
import jax.numpy as jnp
from jax import lax
from jax.experimental import pallas as pl
from jax.experimental.pallas import tpu as pltpu
from jax.experimental.pallas import tpu_sc as plsc

BATCH = 4
NUM_TOKENS = 8192
EMBED_DIM = 1024

NC = 2
NS = 16
NW = NC * NS
L = 16

TOK_PER_TILE = NUM_TOKENS // NW
T_CH = 16
CH = T_CH * EMBED_DIM
N_CH = TOK_PER_TILE // T_CH
NBUF = 5
LOOKAHEAD = 3
N_IT = N_CH * BATCH
UNROLL = 8


def _body(tok_hbm, tbl_hbm, out_hbm, *scratch):
    tbl_v = scratch[0:2]
    tok_v = scratch[2:2 + NBUF]
    s_tbl = scratch[2 + NBUF:4 + NBUF]
    s_in = scratch[4 + NBUF:4 + 2 * NBUF]
    s_out = scratch[4 + 2 * NBUF:4 + 3 * NBUF]

    wid = lax.axis_index("s") * NC + lax.axis_index("c")
    t_base = wid * TOK_PER_TILE

    def row0(ci):
        return t_base + ci * T_CH

    def start_tbl(ci):
        return pltpu.async_copy(
            tbl_hbm.at[pl.ds(row0(ci), T_CH), :], tbl_v[ci % 2], s_tbl[ci % 2])

    def start_in(g):
        ci, b = divmod(g, BATCH)
        return pltpu.async_copy(
            tok_hbm.at[b, pl.ds(row0(ci), T_CH), :], tok_v[g % NBUF], s_in[g % NBUF])

    def start_out(g):
        ci, b = divmod(g, BATCH)
        return pltpu.async_copy(
            tok_v[g % NBUF], out_hbm.at[b, pl.ds(row0(ci), T_CH), :], s_out[g % NBUF])

    tbl_d = {ci: start_tbl(ci) for ci in range(min(2, N_CH))}
    in_d = {g: start_in(g) for g in range(min(NBUF, N_IT))}
    out_d = {}

    for g in range(N_IT):
        ci, b = divmod(g, BATCH)
        tb = ci % 2
        buf = g % NBUF

        if b == 0:
            tbl_d.pop(ci).wait()

        h = g + LOOKAHEAD
        if NBUF <= h < N_IT:
            out_d.pop(h - NBUF).wait()
            in_d[h] = start_in(h)

        in_d.pop(g).wait()

        tok_b = tok_v[buf]
        tbl_b = tbl_v[tb]

        @plsc.parallel_loop(0, CH, L, unroll=UNROLL)
        def add_body(i):
            r = lax.shift_right_logical(i, 10)
            c = pl.multiple_of(lax.bitwise_and(i, EMBED_DIM - 1), L)
            tok_b[r, pl.ds(c, L)] = tok_b[r, pl.ds(c, L)] + tbl_b[r, pl.ds(c, L)]

        out_d[g] = start_out(g)

        if b == BATCH - 1 and ci + 2 < N_CH:
            tbl_d[ci + 2] = start_tbl(ci + 2)

    for od in out_d.values():
        od.wait()


@jax.jit
def _pos_add(encoded_tokens, position_table):
    mesh = plsc.VectorSubcoreMesh(core_axis_name="c", subcore_axis_name="s")
    return pl.kernel(
        _body,
        out_type=jax.ShapeDtypeStruct((BATCH, NUM_TOKENS, EMBED_DIM), jnp.float32),
        mesh=mesh,
        compiler_params=pltpu.CompilerParams(use_tc_tiling_on_sc=True),
        scratch_types=(
            [pltpu.VMEM((T_CH, EMBED_DIM), jnp.float32)] * 2
            + [pltpu.VMEM((T_CH, EMBED_DIM), jnp.float32)] * NBUF
            + [pltpu.SemaphoreType.DMA] * (2 + 2 * NBUF)
        ),
    )(encoded_tokens, position_table)


def kernel(encoded_tokens, position_table):
    return _pos_add(encoded_tokens, position_table)
